# Initial kernel scaffold; baseline (speedup 1.0000x reference)
import jax
import jax.numpy as jnp
from jax.experimental import pallas as pl


def kernel(mem, data, W, b, noise, d_idx):
    raise NotImplementedError("write your pallas kernel here")


# speedup vs baseline: 1.0028x; 1.0028x over previous
import jax
import jax.numpy as jnp
from jax.experimental import pallas as pl

LV_LR = 0.01
SIGMA = 1.0
NOISE_SCALE = (2.0 * LV_LR) ** 0.5


def _update_body(lv_ref, dr_ref, nz_ref, w_ref, b_ref, out_ref):
    lv = lv_ref[...]
    w = w_ref[...]
    pred = jnp.dot(lv, w, preferred_element_type=jnp.float32) + b_ref[...]
    resid = dr_ref[...] - pred
    g = jax.lax.dot_general(
        resid, w, (((1,), (1,)), ((), ())), preferred_element_type=jnp.float32
    ) - lv
    out_ref[...] = LV_LR * g + NOISE_SCALE * nz_ref[...]


def kernel(mem, data, W, b, noise, d_idx):
    P, N, D = mem.shape
    B, DD = data.shape
    R = B * P
    d_rep = jnp.repeat(d_idx, P)
    p_idx = jnp.tile(jnp.arange(P, dtype=d_idx.dtype), B)
    lv = mem.reshape(P * N, D)[p_idx * N + d_rep]
    data_rep = jnp.repeat(data, P, axis=0)
    TR = 2048
    update = pl.pallas_call(
        _update_body,
        grid=(R // TR,),
        in_specs=[
            pl.BlockSpec((TR, D), lambda i: (i, 0)),
            pl.BlockSpec((TR, DD), lambda i: (i, 0)),
            pl.BlockSpec((TR, D), lambda i: (i, 0)),
            pl.BlockSpec((D, DD), lambda i: (0, 0)),
            pl.BlockSpec((1, DD), lambda i: (0, 0)),
        ],
        out_specs=pl.BlockSpec((TR, D), lambda i: (i, 0)),
        out_shape=jax.ShapeDtypeStruct((R, D), jnp.float32),
    )(lv, data_rep, noise, W, b.reshape(1, DD))
    return mem.at[p_idx, d_rep].add(update)
